# baseline (speedup 1.0000x reference)
import functools

import jax
import jax.numpy as jnp
from jax import lax
from jax.experimental import pallas as pl
from jax.experimental.pallas import tpu as pltpu
from jax.experimental.pallas import tpu_sc as plsc

N = 65536
K = 1024
D = 64
BT = 1024
GRID = N // BT
COMMITMENT_COST = 0.25

NW = 32
BPW = N // NW
CH = 128
NCH = BPW // CH


def _tc_body(z_ref, cb_ref, idx_ref, loss_ref, plex_ref,
             counts_acc, e2_acc, iota_acc, msum_acc):
    step = pl.program_id(0)
    zb = z_ref[...]
    cb = cb_ref[...]

    @pl.when(step == 0)
    def _init():
        e2_acc[...] = jnp.sum(cb * cb, axis=1)[None, :]
        iota_acc[...] = lax.broadcasted_iota(jnp.int32, (1, K), 1)
        counts_acc[...] = jnp.zeros_like(counts_acc)
        msum_acc[0, 0] = 0.0

    z2 = jnp.sum(zb * zb, axis=1, keepdims=True)
    mm2 = lax.dot_general(zb + zb, cb, (((1,), (1,)), ((), ())))
    d = (z2 + e2_acc[...]) - mm2
    dmin = jnp.min(d, axis=1)
    mask = d <= dmin[:, None]
    idx = jnp.min(jnp.where(mask, iota_acc[...], K), axis=1)
    idx_ref[...] = idx.astype(jnp.int32)
    counts_acc[...] += jnp.sum(mask.astype(jnp.float32), axis=0, keepdims=True)
    msum_acc[0, 0] += jnp.sum(dmin)

    @pl.when(step == GRID - 1)
    def _finalize():
        ql = msum_acc[0, 0] / jnp.float32(N * D)
        loss_ref[...] = jnp.stack([COMMITMENT_COST * ql, ql]).reshape(1, 2)
        p = counts_acc[0, :] * jnp.float32(1.0 / N)
        plex = jnp.exp(-jnp.sum(p * jnp.log(p + 1e-10)))
        plex_ref[...] = plex.reshape(1, 1)


def _tc_call(z, codebook, interpret=False):
    return pl.pallas_call(
        _tc_body,
        grid=(GRID,),
        in_specs=[
            pl.BlockSpec((BT, D), lambda i: (i, 0)),
            pl.BlockSpec((K, D), lambda i: (0, 0)),
        ],
        out_specs=[
            pl.BlockSpec((BT,), lambda i: (i,)),
            pl.BlockSpec((1, 2), lambda i: (0, 0)),
            pl.BlockSpec((1, 1), lambda i: (0, 0)),
        ],
        out_shape=[
            jax.ShapeDtypeStruct((N,), jnp.int32),
            jax.ShapeDtypeStruct((1, 2), jnp.float32),
            jax.ShapeDtypeStruct((1, 1), jnp.float32),
        ],
        scratch_shapes=[
            pltpu.VMEM((1, K), jnp.float32),
            pltpu.VMEM((1, K), jnp.float32),
            pltpu.VMEM((1, K), jnp.int32),
            pltpu.SMEM((1, 1), jnp.float32),
        ],
        interpret=interpret,
    )(z, codebook)


def _sc_gather_body(cb_hbm, idx_hbm, out_hbm, idx_v, buf0, buf1, sem0, sem1):
    wid = lax.axis_index("s") * 2 + lax.axis_index("c")
    base = wid * BPW
    pltpu.sync_copy(idx_hbm.at[pl.ds(base, BPW)], idx_v)
    bufs = (buf0, buf1)
    sems = (sem0, sem1)
    cps = [None, None]
    cps[0] = pltpu.async_copy(cb_hbm.at[idx_v.at[pl.ds(0, CH)]], buf0, sem0)
    for c in range(NCH):
        nxt = c + 1
        if nxt < NCH:
            cps[nxt % 2] = pltpu.async_copy(
                cb_hbm.at[idx_v.at[pl.ds(nxt * CH, CH)]],
                bufs[nxt % 2], sems[nxt % 2])
        cps[c % 2].wait()
        pltpu.sync_copy(bufs[c % 2], out_hbm.at[pl.ds(base + c * CH, CH)])


@functools.cache
def _sc_gather():
    return pl.kernel(
        _sc_gather_body,
        mesh=plsc.VectorSubcoreMesh(core_axis_name="c", subcore_axis_name="s"),
        compiler_params=pltpu.CompilerParams(use_tc_tiling_on_sc=False),
        out_type=jax.ShapeDtypeStruct((N, D), jnp.float32),
        scratch_types=[
            pltpu.VMEM((BPW,), jnp.int32),
            pltpu.VMEM((CH, D), jnp.float32),
            pltpu.VMEM((CH, D), jnp.float32),
            pltpu.SemaphoreType.DMA,
            pltpu.SemaphoreType.DMA,
        ],
    )


def kernel(z, codebook):
    idx, losses, plex = _tc_call(z, codebook)
    quantized = _sc_gather()(codebook, idx)
    commitment_loss = losses[0, 0]
    q_latent_loss = losses[0, 1]
    perplexity = plex[0, 0]
    return quantized, commitment_loss, q_latent_loss, perplexity, idx


# speedup vs baseline: 1.2529x; 1.2529x over previous
import functools

import jax
import jax.numpy as jnp
from jax import lax
from jax.experimental import pallas as pl
from jax.experimental.pallas import tpu as pltpu
from jax.experimental.pallas import tpu_sc as plsc

N = 65536
K = 1024
D = 64
BT = 1024
GRID = N // BT
COMMITMENT_COST = 0.25

NW = 32
BPW = N // NW
CH = 128
NCH = BPW // CH


def _tc_body(z_ref, cb_ref, idx_ref, loss_ref, plex_ref,
             counts_acc, e2_acc, iota_acc, msum_acc):
    step = pl.program_id(0)
    zb = z_ref[...]
    cb = cb_ref[...]

    @pl.when(step == 0)
    def _init():
        e2_acc[...] = jnp.sum(cb * cb, axis=1)[None, :]
        iota_acc[...] = lax.broadcasted_iota(
            jnp.int32, (1, K), 1).astype(jnp.float32)
        counts_acc[...] = jnp.zeros_like(counts_acc)
        msum_acc[0, 0] = 0.0

    z2 = jnp.sum(zb * zb, axis=1, keepdims=True)
    mm2 = lax.dot_general(zb + zb, cb, (((1,), (1,)), ((), ())))
    d = (z2 + e2_acc[...]) - mm2
    dmin = jnp.min(d, axis=1, keepdims=True)
    mask = d <= dmin
    idxf = jnp.min(jnp.where(mask, iota_acc[...], jnp.float32(K)),
                   axis=1, keepdims=True)
    idx_ref[...] = idxf.astype(jnp.int32)
    counts_acc[...] += jnp.sum(mask.astype(jnp.float32), axis=0, keepdims=True)
    msum_acc[0, 0] += jnp.sum(dmin)

    @pl.when(step == GRID - 1)
    def _finalize():
        ql = msum_acc[0, 0] / jnp.float32(N * D)
        loss_ref[...] = jnp.stack([COMMITMENT_COST * ql, ql]).reshape(1, 2)
        p = counts_acc[0, :] * jnp.float32(1.0 / N)
        plex = jnp.exp(-jnp.sum(p * jnp.log(p + 1e-10)))
        plex_ref[...] = plex.reshape(1, 1)


def _tc_call(z, codebook, interpret=False):
    return pl.pallas_call(
        _tc_body,
        grid=(GRID,),
        in_specs=[
            pl.BlockSpec((BT, D), lambda i: (i, 0)),
            pl.BlockSpec((K, D), lambda i: (0, 0)),
        ],
        out_specs=[
            pl.BlockSpec((BT, 1), lambda i: (i, 0)),
            pl.BlockSpec((1, 2), lambda i: (0, 0)),
            pl.BlockSpec((1, 1), lambda i: (0, 0)),
        ],
        out_shape=[
            jax.ShapeDtypeStruct((N, 1), jnp.int32),
            jax.ShapeDtypeStruct((1, 2), jnp.float32),
            jax.ShapeDtypeStruct((1, 1), jnp.float32),
        ],
        scratch_shapes=[
            pltpu.VMEM((1, K), jnp.float32),
            pltpu.VMEM((1, K), jnp.float32),
            pltpu.VMEM((1, K), jnp.float32),
            pltpu.SMEM((1, 1), jnp.float32),
        ],
        interpret=interpret,
    )(z, codebook)


def _sc_gather_body(cb_hbm, idx_hbm, out_hbm, idx_v, buf0, buf1, sem0, sem1):
    wid = lax.axis_index("s") * 2 + lax.axis_index("c")
    base = wid * BPW
    pltpu.sync_copy(idx_hbm.at[pl.ds(base, BPW)], idx_v)
    bufs = (buf0, buf1)
    sems = (sem0, sem1)
    cps = [None, None]
    cps[0] = pltpu.async_copy(cb_hbm.at[idx_v.at[pl.ds(0, CH)]], buf0, sem0)
    for c in range(NCH):
        nxt = c + 1
        if nxt < NCH:
            cps[nxt % 2] = pltpu.async_copy(
                cb_hbm.at[idx_v.at[pl.ds(nxt * CH, CH)]],
                bufs[nxt % 2], sems[nxt % 2])
        cps[c % 2].wait()
        pltpu.sync_copy(bufs[c % 2], out_hbm.at[pl.ds(base + c * CH, CH)])


@functools.cache
def _sc_gather():
    return pl.kernel(
        _sc_gather_body,
        mesh=plsc.VectorSubcoreMesh(core_axis_name="c", subcore_axis_name="s"),
        compiler_params=pltpu.CompilerParams(use_tc_tiling_on_sc=False),
        out_type=jax.ShapeDtypeStruct((N, D), jnp.float32),
        scratch_types=[
            pltpu.VMEM((BPW,), jnp.int32),
            pltpu.VMEM((CH, D), jnp.float32),
            pltpu.VMEM((CH, D), jnp.float32),
            pltpu.SemaphoreType.DMA,
            pltpu.SemaphoreType.DMA,
        ],
    )


def kernel(z, codebook):
    idx_col, losses, plex = _tc_call(z, codebook)
    idx = idx_col.reshape(N)
    quantized = _sc_gather()(codebook, idx)
    commitment_loss = losses[0, 0]
    q_latent_loss = losses[0, 1]
    perplexity = plex[0, 0]
    return quantized, commitment_loss, q_latent_loss, perplexity, idx


# speedup vs baseline: 1.2565x; 1.0028x over previous
import functools

import jax
import jax.numpy as jnp
from jax import lax
from jax.experimental import pallas as pl
from jax.experimental.pallas import tpu as pltpu
from jax.experimental.pallas import tpu_sc as plsc

N = 65536
K = 1024
D = 64
BT = 1024
GRID = N // BT
COMMITMENT_COST = 0.25

NW = 32
BPW = N // NW
CH = 128
NCH = BPW // CH


def _tc_body(z_ref, cb_ref, idx_ref, loss_ref, plex_ref,
             counts_acc, e2_acc, iota_acc, msum_acc):
    step = pl.program_id(0)
    zb = z_ref[...]
    cb = cb_ref[...]

    @pl.when(step == 0)
    def _init():
        e2_acc[...] = jnp.sum(cb * cb, axis=1)[None, :]
        iota_acc[...] = lax.broadcasted_iota(
            jnp.int32, (1, K), 1).astype(jnp.float32)
        counts_acc[...] = jnp.zeros_like(counts_acc)
        msum_acc[0, 0] = 0.0

    z2 = jnp.sum(zb * zb, axis=1, keepdims=True)
    mm2 = lax.dot_general(zb + zb, cb, (((1,), (1,)), ((), ())))
    d = (z2 + e2_acc[...]) - mm2
    dmin = jnp.min(d, axis=1, keepdims=True)
    mask = d <= dmin
    idxf = jnp.min(jnp.where(mask, iota_acc[...], jnp.float32(K)),
                   axis=1, keepdims=True)
    idx_ref[...] = idxf.astype(jnp.int32)
    counts_acc[...] += jnp.sum(mask.astype(jnp.float32), axis=0, keepdims=True)
    msum_acc[0, 0] += jnp.sum(dmin)

    @pl.when(step == GRID - 1)
    def _finalize():
        ql = msum_acc[0, 0] / jnp.float32(N * D)
        loss_ref[...] = jnp.stack([COMMITMENT_COST * ql, ql]).reshape(1, 2)
        p = counts_acc[0, :] * jnp.float32(1.0 / N)
        plex = jnp.exp(-jnp.sum(p * jnp.log(p + 1e-10)))
        plex_ref[...] = plex.reshape(1, 1)


def _tc_call(z, codebook, interpret=False):
    return pl.pallas_call(
        _tc_body,
        grid=(GRID,),
        in_specs=[
            pl.BlockSpec((BT, D), lambda i: (i, 0)),
            pl.BlockSpec((K, D), lambda i: (0, 0)),
        ],
        out_specs=[
            pl.BlockSpec((BT, 1), lambda i: (i, 0)),
            pl.BlockSpec((1, 2), lambda i: (0, 0)),
            pl.BlockSpec((1, 1), lambda i: (0, 0)),
        ],
        out_shape=[
            jax.ShapeDtypeStruct((N, 1), jnp.int32),
            jax.ShapeDtypeStruct((1, 2), jnp.float32),
            jax.ShapeDtypeStruct((1, 1), jnp.float32),
        ],
        scratch_shapes=[
            pltpu.VMEM((1, K), jnp.float32),
            pltpu.VMEM((1, K), jnp.float32),
            pltpu.VMEM((1, K), jnp.float32),
            pltpu.SMEM((1, 1), jnp.float32),
        ],
        interpret=interpret,
    )(z, codebook)


def _sc_gather_body(cb_hbm, idx_hbm, out_hbm, idx_v, buf0, buf1,
                    sem0, sem1, ssem0, ssem1):
    wid = lax.axis_index("s") * 2 + lax.axis_index("c")
    base = wid * BPW
    pltpu.sync_copy(idx_hbm.at[pl.ds(base, BPW)], idx_v)
    bufs = (buf0, buf1)
    sems = (sem0, sem1)
    ssems = (ssem0, ssem1)
    cps = [None, None]
    stcps = [None, None]
    cps[0] = pltpu.async_copy(cb_hbm.at[idx_v.at[pl.ds(0, CH)]], buf0, sem0)
    for c in range(NCH):
        nxt = c + 1
        if nxt < NCH:
            if nxt >= 2:
                stcps[nxt % 2].wait()
            cps[nxt % 2] = pltpu.async_copy(
                cb_hbm.at[idx_v.at[pl.ds(nxt * CH, CH)]],
                bufs[nxt % 2], sems[nxt % 2])
        cps[c % 2].wait()
        stcps[c % 2] = pltpu.async_copy(
            bufs[c % 2], out_hbm.at[pl.ds(base + c * CH, CH)], ssems[c % 2])
    stcps[(NCH - 2) % 2].wait()
    stcps[(NCH - 1) % 2].wait()


@functools.cache
def _sc_gather():
    return pl.kernel(
        _sc_gather_body,
        mesh=plsc.VectorSubcoreMesh(core_axis_name="c", subcore_axis_name="s"),
        compiler_params=pltpu.CompilerParams(use_tc_tiling_on_sc=False),
        out_type=jax.ShapeDtypeStruct((N, D), jnp.float32),
        scratch_types=[
            pltpu.VMEM((BPW,), jnp.int32),
            pltpu.VMEM((CH, D), jnp.float32),
            pltpu.VMEM((CH, D), jnp.float32),
            pltpu.SemaphoreType.DMA,
            pltpu.SemaphoreType.DMA,
            pltpu.SemaphoreType.DMA,
            pltpu.SemaphoreType.DMA,
        ],
    )


def kernel(z, codebook):
    idx_col, losses, plex = _tc_call(z, codebook)
    idx = idx_col.reshape(N)
    quantized = _sc_gather()(codebook, idx)
    commitment_loss = losses[0, 0]
    q_latent_loss = losses[0, 1]
    perplexity = plex[0, 0]
    return quantized, commitment_loss, q_latent_loss, perplexity, idx
